# baseline (speedup 1.0000x reference)
import functools

import jax
import jax.numpy as jnp
from jax import lax
from jax.experimental import pallas as pl
from jax.experimental.pallas import tpu as pltpu
from jax.experimental.pallas import tpu_sc as plsc

NUM_CLASSES = 22
SEQ_LEN = 8192

_info = plsc.get_sparse_core_info()
_NC, _NS, _L = _info.num_cores, _info.num_subcores, _info.num_lanes
_NW = _NC * _NS
_TOK_PER_W = SEQ_LEN // _NW
_VECS = _TOK_PER_W // _L
_CHUNKS = 2
_VPC = _VECS // _CHUNKS
_COLS = _VPC * _L


@functools.partial(
    pl.kernel,
    mesh=plsc.VectorSubcoreMesh(core_axis_name="c", subcore_axis_name="s"),
    out_type=jax.ShapeDtypeStruct((NUM_CLASSES, SEQ_LEN), jnp.float32),
    scratch_types=[
        pltpu.VMEM((_TOK_PER_W,), jnp.int32),
        pltpu.VMEM((NUM_CLASSES, _TOK_PER_W), jnp.float32),
        pltpu.SemaphoreType.DMA,
    ],
)
def _onehot_sc(x_hbm, out_hbm, x_v, blk_v, sem):
    wid = lax.axis_index("s") * _NC + lax.axis_index("c")
    base = wid * _TOK_PER_W
    pltpu.sync_copy(x_hbm.at[pl.ds(base, _TOK_PER_W)], x_v)
    one = jnp.full((_L,), 1.0, dtype=jnp.float32)
    zero = jnp.zeros((_L,), dtype=jnp.float32)
    copies = []
    for h in range(_CHUNKS):
        for j in range(h * _VPC, (h + 1) * _VPC):
            xv = x_v[pl.ds(j * _L, _L)]
            for c in range(NUM_CLASSES):
                blk_v[c, pl.ds(j * _L, _L)] = jnp.where(xv == c, one, zero)
        copies.append(pltpu.async_copy(
            blk_v.at[:, pl.ds(h * _COLS, _COLS)],
            out_hbm.at[:, pl.ds(base + h * _COLS, _COLS)],
            sem,
        ))
    for cp in copies:
        cp.wait()


def kernel(x):
    return _onehot_sc(x.astype(jnp.int32)).reshape(1, NUM_CLASSES, SEQ_LEN)
